# Initial kernel scaffold; baseline (speedup 1.0000x reference)
import jax
import jax.numpy as jnp
from jax.experimental import pallas as pl


def kernel(x, edge_index, W1, b1, g1, be1, p1_Wroot, p1_Wrel, p1_brel, W2, b2, g2, be2, p2_Wroot, p2_Wrel, p2_brel, Wf, bf):
    raise NotImplementedError("write your pallas kernel here")


# speedup vs baseline: 1.0499x; 1.0499x over previous
import functools
import jax
import jax.numpy as jnp
from jax.experimental import pallas as pl

_F32 = jnp.float32


def _mm_body(x_ref, w_ref, o_ref):
    o_ref[:] = jnp.dot(x_ref[:], w_ref[:], preferred_element_type=_F32)


def _matmul(x, w, bn=256):
    n, din = x.shape
    dout = w.shape[1]
    grid = (n + bn - 1) // bn
    return pl.pallas_call(
        _mm_body,
        grid=(grid,),
        in_specs=[
            pl.BlockSpec((bn, din), lambda i: (i, 0)),
            pl.BlockSpec((din, dout), lambda i: (0, 0)),
        ],
        out_specs=pl.BlockSpec((bn, dout), lambda i: (i, 0)),
        out_shape=jax.ShapeDtypeStruct((n, dout), _F32),
    )(x, w)


def _ln_relu_body(x_ref, b_ref, g_ref, be_ref, o_ref):
    t = x_ref[:] + b_ref[:]
    mu = jnp.mean(t, axis=-1, keepdims=True)
    var = jnp.mean((t - mu) ** 2, axis=-1, keepdims=True)
    y = (t - mu) * jax.lax.rsqrt(var + 1e-5) * g_ref[:] + be_ref[:]
    o_ref[:] = jnp.maximum(y, 0.0)


def _ln_relu(x, b, g, be, bn=256):
    n, d = x.shape
    grid = (n + bn - 1) // bn
    vec = pl.BlockSpec((1, d), lambda i: (0, 0))
    return pl.pallas_call(
        _ln_relu_body,
        grid=(grid,),
        in_specs=[pl.BlockSpec((bn, d), lambda i: (i, 0)), vec, vec, vec],
        out_specs=pl.BlockSpec((bn, d), lambda i: (i, 0)),
        out_shape=jax.ShapeDtypeStruct((n, d), _F32),
    )(x, b.reshape(1, d), g.reshape(1, d), be.reshape(1, d))


def _score_body(agg_ref, h_ref, wrel_ref, wroot_ref, brel_ref, o_ref):
    s = jnp.dot(agg_ref[:], wrel_ref[:], preferred_element_type=_F32)
    s += jnp.dot(h_ref[:], wroot_ref[:], preferred_element_type=_F32)
    o_ref[:] = s + brel_ref[:]


def _score(agg, h, wrel, wroot, brel, bn=256):
    n, d = agg.shape
    grid = (n + bn - 1) // bn
    wrel_p = jnp.pad(wrel, ((0, 0), (0, 127)))
    wroot_p = jnp.pad(wroot, ((0, 0), (0, 127)))
    brel_p = jnp.pad(brel.reshape(1, 1), ((0, 0), (0, 127)))
    out = pl.pallas_call(
        _score_body,
        grid=(grid,),
        in_specs=[
            pl.BlockSpec((bn, d), lambda i: (i, 0)),
            pl.BlockSpec((bn, d), lambda i: (i, 0)),
            pl.BlockSpec((d, 128), lambda i: (0, 0)),
            pl.BlockSpec((d, 128), lambda i: (0, 0)),
            pl.BlockSpec((1, 128), lambda i: (0, 0)),
        ],
        out_specs=pl.BlockSpec((bn, 128), lambda i: (i, 0)),
        out_shape=jax.ShapeDtypeStruct((n, 128), _F32),
    )(agg, h, wrel_p, wroot_p, brel_p)
    return out[:, 0]


def _pool_final_body(x_ref, wf_ref, bf_ref, o_ref):
    pooled = jnp.mean(x_ref[:], axis=0, keepdims=True)
    o_ref[:] = jnp.dot(pooled, wf_ref[:], preferred_element_type=_F32) + bf_ref[:]


def _pool_final(x, wf, bf):
    n, d = x.shape
    dout = wf.shape[1]
    return pl.pallas_call(
        _pool_final_body,
        in_specs=[
            pl.BlockSpec((n, d), lambda: (0, 0)),
            pl.BlockSpec((d, dout), lambda: (0, 0)),
            pl.BlockSpec((1, dout), lambda: (0, 0)),
        ],
        out_specs=pl.BlockSpec((1, dout), lambda: (0, 0)),
        out_shape=jax.ShapeDtypeStruct((1, dout), _F32),
    )(x, wf, bf.reshape(1, dout))


def _gcn(x, W, b, g, be, src, dst, emask):
    n = x.shape[0]
    h = _matmul(x, W)
    loop = jnp.arange(n, dtype=src.dtype)
    s = jnp.concatenate([src, loop])
    d = jnp.concatenate([dst, loop])
    ew = jnp.concatenate([emask, 2.0 * jnp.ones((n,), x.dtype)])
    deg = jax.ops.segment_sum(ew, d, num_segments=n)
    dinv = jnp.where(deg > 0, jax.lax.rsqrt(deg), 0.0)
    norm = dinv[s] * ew * dinv[d]
    out = jnp.zeros_like(h).at[d].add(h[s] * norm[:, None])
    return _ln_relu(out, b, g, be)


def _sag_pool(x, src, dst, emask, Wroot, Wrel, brel, k):
    n = x.shape[0]
    agg = jnp.zeros((n, x.shape[1]), x.dtype).at[dst].add(x[src] * emask[:, None])
    score = _score(agg, x, Wrel, Wroot, brel)
    vals, perm = jax.lax.top_k(score, k)
    xn = x[perm] * jnp.tanh(vals)[:, None]
    new_id = jnp.full((n,), -1, jnp.int32).at[perm].set(jnp.arange(k, dtype=jnp.int32))
    ns = new_id[src]
    nd = new_id[dst]
    valid = (ns >= 0) & (nd >= 0) & (emask > 0)
    ns = jnp.where(valid, ns, 0)
    nd = jnp.where(valid, nd, 0)
    return xn, ns, nd, valid.astype(x.dtype)


def kernel(x, edge_index, W1, b1, g1, be1, p1_Wroot, p1_Wrel, p1_brel, W2, b2, g2, be2, p2_Wroot, p2_Wrel, p2_brel, Wf, bf):
    src = edge_index[0].astype(jnp.int32)
    dst = edge_index[1].astype(jnp.int32)
    m0 = jnp.ones((src.shape[0],), x.dtype)
    h = _gcn(x, W1, b1, g1, be1, src, dst, m0)
    k1 = (x.shape[0] + 1) // 2
    h, s1, d1, m1 = _sag_pool(h, src, dst, m0, p1_Wroot, p1_Wrel, p1_brel, k1)
    h = _gcn(h, W2, b2, g2, be2, s1, d1, m1)
    k2 = (k1 + 1) // 2
    h, s2, d2, m2 = _sag_pool(h, s1, d1, m1, p2_Wroot, p2_Wrel, p2_brel, k2)
    return _pool_final(h, Wf, bf)


# speedup vs baseline: 1.9545x; 1.8616x over previous
import jax
import jax.numpy as jnp
from jax.experimental import pallas as pl

_F32 = jnp.float32


def _mm_body(x_ref, w_ref, o_ref):
    o_ref[:] = jnp.dot(x_ref[:], w_ref[:], preferred_element_type=_F32)


def _matmul(x, w, bn=256):
    n, din = x.shape
    dout = w.shape[1]
    grid = (n + bn - 1) // bn
    return pl.pallas_call(
        _mm_body,
        grid=(grid,),
        in_specs=[
            pl.BlockSpec((bn, din), lambda i: (i, 0)),
            pl.BlockSpec((din, dout), lambda i: (0, 0)),
        ],
        out_specs=pl.BlockSpec((bn, dout), lambda i: (i, 0)),
        out_shape=jax.ShapeDtypeStruct((n, dout), _F32),
    )(x, w)


def _gcn_post_body(raw_ref, h_ref, dinv_ref, b_ref, g_ref, be_ref, o_ref):
    di = dinv_ref[:]
    t = raw_ref[:] * di + 2.0 * di * di * h_ref[:] + b_ref[:]
    mu = jnp.mean(t, axis=-1, keepdims=True)
    var = jnp.mean((t - mu) ** 2, axis=-1, keepdims=True)
    y = (t - mu) * jax.lax.rsqrt(var + 1e-5) * g_ref[:] + be_ref[:]
    o_ref[:] = jnp.maximum(y, 0.0)


def _gcn_post(raw, h, dinv, b, g, be, bn=256):
    n, d = raw.shape
    grid = (n + bn - 1) // bn
    vec = pl.BlockSpec((1, d), lambda i: (0, 0))
    return pl.pallas_call(
        _gcn_post_body,
        grid=(grid,),
        in_specs=[
            pl.BlockSpec((bn, d), lambda i: (i, 0)),
            pl.BlockSpec((bn, d), lambda i: (i, 0)),
            pl.BlockSpec((bn, 1), lambda i: (i, 0)),
            vec, vec, vec,
        ],
        out_specs=pl.BlockSpec((bn, d), lambda i: (i, 0)),
        out_shape=jax.ShapeDtypeStruct((n, d), _F32),
    )(raw, h, dinv.reshape(n, 1), b.reshape(1, d), g.reshape(1, d), be.reshape(1, d))


def _score_body(agg_ref, h_ref, wrel_ref, wroot_ref, brel_ref, o_ref):
    s = jnp.dot(agg_ref[:], wrel_ref[:], preferred_element_type=_F32)
    s += jnp.dot(h_ref[:], wroot_ref[:], preferred_element_type=_F32)
    o_ref[:] = s + brel_ref[:]


def _score(agg, h, wrel, wroot, brel, bn=256):
    n, d = agg.shape
    grid = (n + bn - 1) // bn
    wrel_p = jnp.pad(wrel, ((0, 0), (0, 127)))
    wroot_p = jnp.pad(wroot, ((0, 0), (0, 127)))
    brel_p = jnp.pad(brel.reshape(1, 1), ((0, 0), (0, 127)))
    out = pl.pallas_call(
        _score_body,
        grid=(grid,),
        in_specs=[
            pl.BlockSpec((bn, d), lambda i: (i, 0)),
            pl.BlockSpec((bn, d), lambda i: (i, 0)),
            pl.BlockSpec((d, 128), lambda i: (0, 0)),
            pl.BlockSpec((d, 128), lambda i: (0, 0)),
            pl.BlockSpec((1, 128), lambda i: (0, 0)),
        ],
        out_specs=pl.BlockSpec((bn, 128), lambda i: (i, 0)),
        out_shape=jax.ShapeDtypeStruct((n, 128), _F32),
    )(agg, h, wrel_p, wroot_p, brel_p)
    return out[:, 0]


def _pool_final_body(x_ref, wf_ref, bf_ref, o_ref):
    pooled = jnp.mean(x_ref[:], axis=0, keepdims=True)
    o_ref[:] = jnp.dot(pooled, wf_ref[:], preferred_element_type=_F32) + bf_ref[:]


def _pool_final(x, wf, bf):
    n, d = x.shape
    dout = wf.shape[1]
    return pl.pallas_call(
        _pool_final_body,
        in_specs=[
            pl.BlockSpec((n, d), lambda: (0, 0)),
            pl.BlockSpec((d, dout), lambda: (0, 0)),
            pl.BlockSpec((1, dout), lambda: (0, 0)),
        ],
        out_specs=pl.BlockSpec((1, dout), lambda: (0, 0)),
        out_shape=jax.ShapeDtypeStruct((1, dout), _F32),
    )(x, wf, bf.reshape(1, dout))


def _gcn(x, W, b, g, be, sw, dw, ecount):
    n = x.shape[0]
    h = _matmul(x, W)
    deg = jax.ops.segment_sum(ecount, dw, num_segments=n + 1)[:n] + 2.0
    dinv = jax.lax.rsqrt(deg)
    hs = h * dinv[:, None]
    raw = jnp.zeros((n + 1, h.shape[1]), _F32).at[dw].add(hs[sw])[:n]
    return h, _gcn_post(raw, h, dinv, b, g, be)


def _sag_pool(x, src, dst, emask, sw, dw, Wroot, Wrel, brel, k):
    n = x.shape[0]
    agg = jnp.zeros((n + 1, x.shape[1]), x.dtype).at[dw].add(x[sw])[:n]
    score = _score(agg, x, Wrel, Wroot, brel)
    vals, perm = jax.lax.top_k(score, k)
    xn = x[perm] * jnp.tanh(vals)[:, None]
    new_id = jnp.full((n,), -1, jnp.int32).at[perm].set(jnp.arange(k, dtype=jnp.int32))
    ns = new_id[src]
    nd = new_id[dst]
    valid = (ns >= 0) & (nd >= 0) & (emask > 0)
    ns = jnp.where(valid, ns, 0)
    nd = jnp.where(valid, nd, 0)
    return xn, ns, nd, valid.astype(x.dtype)


def kernel(x, edge_index, W1, b1, g1, be1, p1_Wroot, p1_Wrel, p1_brel, W2, b2, g2, be2, p2_Wroot, p2_Wrel, p2_brel, Wf, bf):
    n = x.shape[0]
    src = edge_index[0].astype(jnp.int32)
    dst = edge_index[1].astype(jnp.int32)
    e = src.shape[0]
    m0 = jnp.ones((e,), x.dtype)

    _, h = _gcn(x, W1, b1, g1, be1, src, dst, m0)
    k1 = (n + 1) // 2
    h, s1, d1, m1 = _sag_pool(h, src, dst, m0, src, dst, p1_Wroot, p1_Wrel, p1_brel, k1)

    valid1 = m1 > 0
    sw1 = jnp.where(valid1, s1, 0)
    dw1 = jnp.where(valid1, d1, k1)
    _, h = _gcn(h, W2, b2, g2, be2, sw1, dw1, m1)
    k2 = (k1 + 1) // 2
    h, s2, d2, m2 = _sag_pool(h, s1, d1, m1, sw1, dw1, p2_Wroot, p2_Wrel, p2_brel, k2)

    return _pool_final(h, Wf, bf)


# speedup vs baseline: 1.9660x; 1.0059x over previous
import jax
import jax.numpy as jnp
from jax.experimental import pallas as pl

_F32 = jnp.float32


def _mm_body(x_ref, w_ref, s_ref, o_ref, os_ref):
    h = jnp.dot(x_ref[:], w_ref[:], preferred_element_type=_F32)
    o_ref[:] = h
    os_ref[:] = h * s_ref[:]


def _matmul(x, w, scale, bn=256):
    n, din = x.shape
    dout = w.shape[1]
    grid = (n + bn - 1) // bn
    blk = pl.BlockSpec((bn, dout), lambda i: (i, 0))
    return pl.pallas_call(
        _mm_body,
        grid=(grid,),
        in_specs=[
            pl.BlockSpec((bn, din), lambda i: (i, 0)),
            pl.BlockSpec((din, dout), lambda i: (0, 0)),
            pl.BlockSpec((bn, 1), lambda i: (i, 0)),
        ],
        out_specs=(blk, blk),
        out_shape=(
            jax.ShapeDtypeStruct((n, dout), _F32),
            jax.ShapeDtypeStruct((n, dout), _F32),
        ),
    )(x, w, scale.reshape(n, 1))


def _gcn_post_body(raw_ref, h_ref, dinv_ref, b_ref, g_ref, be_ref, o_ref):
    di = dinv_ref[:]
    t = raw_ref[:] * di + 2.0 * di * di * h_ref[:] + b_ref[:]
    mu = jnp.mean(t, axis=-1, keepdims=True)
    var = jnp.mean((t - mu) ** 2, axis=-1, keepdims=True)
    y = (t - mu) * jax.lax.rsqrt(var + 1e-5) * g_ref[:] + be_ref[:]
    o_ref[:] = jnp.maximum(y, 0.0)


def _gcn_post(raw, h, dinv, b, g, be, bn=256):
    n, d = raw.shape
    grid = (n + bn - 1) // bn
    vec = pl.BlockSpec((1, d), lambda i: (0, 0))
    return pl.pallas_call(
        _gcn_post_body,
        grid=(grid,),
        in_specs=[
            pl.BlockSpec((bn, d), lambda i: (i, 0)),
            pl.BlockSpec((bn, d), lambda i: (i, 0)),
            pl.BlockSpec((bn, 1), lambda i: (i, 0)),
            vec, vec, vec,
        ],
        out_specs=pl.BlockSpec((bn, d), lambda i: (i, 0)),
        out_shape=jax.ShapeDtypeStruct((n, d), _F32),
    )(raw, h, dinv.reshape(n, 1), b.reshape(1, d), g.reshape(1, d), be.reshape(1, d))


def _score_body(agg_ref, h_ref, wrel_ref, wroot_ref, brel_ref, o_ref):
    s = jnp.dot(agg_ref[:], wrel_ref[:], preferred_element_type=_F32)
    s += jnp.dot(h_ref[:], wroot_ref[:], preferred_element_type=_F32)
    o_ref[:] = s + brel_ref[:]


def _score(agg, h, wrel, wroot, brel, bn=256):
    n, d = agg.shape
    grid = (n + bn - 1) // bn
    wrel_p = jnp.pad(wrel, ((0, 0), (0, 127)))
    wroot_p = jnp.pad(wroot, ((0, 0), (0, 127)))
    brel_p = jnp.pad(brel.reshape(1, 1), ((0, 0), (0, 127)))
    out = pl.pallas_call(
        _score_body,
        grid=(grid,),
        in_specs=[
            pl.BlockSpec((bn, d), lambda i: (i, 0)),
            pl.BlockSpec((bn, d), lambda i: (i, 0)),
            pl.BlockSpec((d, 128), lambda i: (0, 0)),
            pl.BlockSpec((d, 128), lambda i: (0, 0)),
            pl.BlockSpec((1, 128), lambda i: (0, 0)),
        ],
        out_specs=pl.BlockSpec((bn, 128), lambda i: (i, 0)),
        out_shape=jax.ShapeDtypeStruct((n, 128), _F32),
    )(agg, h, wrel_p, wroot_p, brel_p)
    return out[:, 0]


def _pool_final_body(x_ref, wf_ref, bf_ref, o_ref):
    pooled = jnp.mean(x_ref[:], axis=0, keepdims=True)
    o_ref[:] = jnp.dot(pooled, wf_ref[:], preferred_element_type=_F32) + bf_ref[:]


def _pool_final(x, wf, bf):
    n, d = x.shape
    dout = wf.shape[1]
    return pl.pallas_call(
        _pool_final_body,
        in_specs=[
            pl.BlockSpec((n, d), lambda: (0, 0)),
            pl.BlockSpec((d, dout), lambda: (0, 0)),
            pl.BlockSpec((1, dout), lambda: (0, 0)),
        ],
        out_specs=pl.BlockSpec((1, dout), lambda: (0, 0)),
        out_shape=jax.ShapeDtypeStruct((1, dout), _F32),
    )(x, wf, bf.reshape(1, dout))


def _gcn(x, W, b, g, be, sw, dw, ecount):
    n = x.shape[0]
    deg = jax.ops.segment_sum(ecount, dw, num_segments=n + 1)[:n] + 2.0
    dinv = jax.lax.rsqrt(deg)
    h, hs = _matmul(x, W, dinv)
    raw = jnp.zeros((n + 1, h.shape[1]), _F32).at[dw].add(hs[sw])[:n]
    return h, _gcn_post(raw, h, dinv, b, g, be)


def _sag_pool(x, src, dst, emask, sw, dw, Wroot, Wrel, brel, k):
    n = x.shape[0]
    agg = jnp.zeros((n + 1, x.shape[1]), x.dtype).at[dw].add(x[sw])[:n]
    score = _score(agg, x, Wrel, Wroot, brel)
    vals, perm = jax.lax.top_k(score, k)
    xn = x[perm] * jnp.tanh(vals)[:, None]
    new_id = jnp.full((n,), -1, jnp.int32).at[perm].set(jnp.arange(k, dtype=jnp.int32))
    ns = new_id[src]
    nd = new_id[dst]
    valid = (ns >= 0) & (nd >= 0) & (emask > 0)
    ns = jnp.where(valid, ns, 0)
    nd = jnp.where(valid, nd, 0)
    return xn, ns, nd, valid.astype(x.dtype)


def kernel(x, edge_index, W1, b1, g1, be1, p1_Wroot, p1_Wrel, p1_brel, W2, b2, g2, be2, p2_Wroot, p2_Wrel, p2_brel, Wf, bf):
    n = x.shape[0]
    src = edge_index[0].astype(jnp.int32)
    dst = edge_index[1].astype(jnp.int32)
    e = src.shape[0]
    m0 = jnp.ones((e,), x.dtype)

    _, h = _gcn(x, W1, b1, g1, be1, src, dst, m0)
    k1 = (n + 1) // 2
    h, s1, d1, m1 = _sag_pool(h, src, dst, m0, src, dst, p1_Wroot, p1_Wrel, p1_brel, k1)

    valid1 = m1 > 0
    sw1 = jnp.where(valid1, s1, 0)
    dw1 = jnp.where(valid1, d1, k1)
    _, h = _gcn(h, W2, b2, g2, be2, sw1, dw1, m1)
    k2 = (k1 + 1) // 2
    h, s2, d2, m2 = _sag_pool(h, s1, d1, m1, sw1, dw1, p2_Wroot, p2_Wrel, p2_brel, k2)

    return _pool_final(h, Wf, bf)
